# Initial kernel scaffold; baseline (speedup 1.0000x reference)
import jax
import jax.numpy as jnp
from jax.experimental import pallas as pl


def kernel(x, edge_index, W1, att_src1, att_dst1, bias1, gamma1, beta1, W2, att_src2, att_dst2, bias2, gamma2, beta2):
    raise NotImplementedError("write your pallas kernel here")


# speedup vs baseline: 44.4228x; 44.4228x over previous
import functools

import jax
import jax.numpy as jnp
from jax import lax
from jax.experimental import pallas as pl
from jax.experimental.pallas import tpu as pltpu
from jax.experimental.pallas import tpu_sc as plsc

_N = 10000
_E = 320000
_NPAD = 10112
_TRASH = _N
_NC, _NS = 2, 16
_NTILES = _NC * _NS
_BE = 128
_ETOT = _E + _N
_KCH = -(-_ETOT // (_NTILES * _BE))
_EPAD = _NTILES * _BE * _KCH
_BN = 1000

_f32 = jnp.float32
_i32 = jnp.int32

_GDN = lax.GatherDimensionNumbers(
    offset_dims=(), collapsed_slice_dims=(0,), start_index_map=(0,))


def _vgather(vec, idx):
    return lax.gather(vec, idx[:, None], _GDN, (1,),
                      mode=lax.GatherScatterMode.PROMISE_IN_BOUNDS)


def _make_edge_kernel(hcols, srow, excol, nck, per_head):
    mesh = plsc.VectorSubcoreMesh(
        core_axis_name="c", subcore_axis_name="s",
        num_cores=_NC, num_subcores=_NS)
    rows_sub = _NPAD // _NS

    @functools.partial(
        pl.kernel,
        out_type=jax.ShapeDtypeStruct((_NC, _NPAD, srow), _f32),
        mesh=mesh,
        compiler_params=pltpu.CompilerParams(use_tc_tiling_on_sc=False),
        scratch_types=[
            pltpu.VMEM((_BE,), _i32),
            pltpu.VMEM((_BE,), _i32),
            pltpu.VMEM((_BE, hcols), _f32),
            pltpu.VMEM((_BE, 16), _f32),
            pltpu.VMEM((_BE, 16), _f32),
            pltpu.VMEM((_BE, srow), _f32),
            pltpu.VMEM((16,), _f32),
            pltpu.VMEM_SHARED((_NPAD, srow), _f32),
            pltpu.SemaphoreType.DMA,
            pltpu.SemaphoreType.DMA,
            pltpu.SemaphoreType.DMA,
        ],
    )
    def edge_kernel(h_hbm, a_hbm, b_hbm, src_hbm, dst_hbm, cvec_hbm, zeros_hbm,
                    out_hbm, sidx, didx, hrows, arows, brows, scat,
                    cvec, acc, sem0, sem1, sem2):
        cid = lax.axis_index("c")
        sid = lax.axis_index("s")
        wid = cid * _NS + sid
        rb = sid * rows_sub
        pltpu.sync_copy(zeros_hbm.at[pl.ds(rb, rows_sub)],
                        acc.at[pl.ds(rb, rows_sub)])
        pltpu.sync_copy(cvec_hbm, cvec)
        plsc.subcore_barrier()
        cv = cvec[...]
        ii = lax.iota(_i32, 16)
        half = lax.shift_right_logical(ii, 3)
        zidx = ii * 0
        scale_idx = [half + 2 * k for k in range(nck)]
        ebase = wid * (_KCH * _BE)

        def chunk(k, carry):
            cb = ebase + k * _BE
            pltpu.sync_copy(src_hbm.at[pl.ds(cb, _BE)], sidx)
            pltpu.sync_copy(dst_hbm.at[pl.ds(cb, _BE)], didx)
            cp0 = pltpu.async_copy(h_hbm.at[sidx], hrows, sem0)
            cp1 = pltpu.async_copy(a_hbm.at[sidx], arows, sem1)
            cp2 = pltpu.async_copy(b_hbm.at[didx], brows, sem2)
            cp0.wait()
            cp1.wait()
            cp2.wait()

            def edge(e, c2):
                av = arows[e, :]
                bv = brows[e, :]
                z = av + bv
                zz = jnp.where(z >= 0, z, 0.2 * z) - cv
                ex = jnp.exp(zz)
                for k2 in range(nck):
                    sc = _vgather(ex, scale_idx[k2]) if per_head else ex
                    hv = hrows[e, pl.ds(k2 * 16, 16)]
                    scat[e, pl.ds(k2 * 16, 16)] = hv * sc
                scat[e, pl.ds(excol, 16)] = ex
                return c2

            lax.fori_loop(0, _BE, edge, 0)
            pltpu.sync_copy(scat, acc.at[didx], add=True)
            return carry

        lax.fori_loop(0, _KCH, chunk, 0)
        plsc.subcore_barrier()
        pltpu.sync_copy(acc.at[pl.ds(rb, rows_sub)],
                        out_hbm.at[cid, pl.ds(rb, rows_sub)])

    return edge_kernel


_edge1 = _make_edge_kernel(hcols=64, srow=80, excol=64, nck=4, per_head=True)
_edge2 = _make_edge_kernel(hcols=48, srow=64, excol=48, nck=3, per_head=False)


def _k1_body(x_ref, w1_ref, ms_ref, md_ref, h_ref, a_ref, b_ref, ca_ref,
             cb_ref):
    i = pl.program_id(0)
    h = jnp.dot(x_ref[...], w1_ref[...], preferred_element_type=_f32)
    h_ref[...] = h
    a = jnp.dot(h, ms_ref[...], preferred_element_type=_f32)
    b = jnp.dot(h, md_ref[...], preferred_element_type=_f32)
    a_ref[...] = a
    b_ref[...] = b
    am = jnp.max(a, axis=0, keepdims=True)
    bm = jnp.max(b, axis=0, keepdims=True)

    @pl.when(i == 0)
    def _():
        ca_ref[...] = am
        cb_ref[...] = bm

    @pl.when(i != 0)
    def _():
        ca_ref[...] = jnp.maximum(ca_ref[...], am)
        cb_ref[...] = jnp.maximum(cb_ref[...], bm)


def _k2_body(acc_ref, kb_ref, bias_ref, g_ref, be_ref, w2_ref, as2_ref,
             bs2_ref, h2_ref, a2_ref, b2_ref, ca_ref, cb_ref):
    i = pl.program_id(0)
    acc = acc_ref[...]
    s = acc[0] + acc[1]
    num = s[:, :64]
    den = s[:, 64:72]
    denf = jnp.dot(den, kb_ref[...], preferred_element_type=_f32)
    o = num / (denf + 1e-16) + bias_ref[...]
    mu = jnp.mean(o, axis=-1, keepdims=True)
    var = jnp.mean((o - mu) ** 2, axis=-1, keepdims=True)
    o = (o - mu) / jnp.sqrt(var + 1e-5) * g_ref[...] + be_ref[...]
    o = jnp.where(o > 0, o, jnp.exp(o) - 1.0)
    h2 = jnp.dot(o, w2_ref[...], preferred_element_type=_f32)
    h2_ref[...] = h2
    a2 = jnp.dot(h2, as2_ref[...], preferred_element_type=_f32)
    b2 = jnp.dot(h2, bs2_ref[...], preferred_element_type=_f32)
    a2_ref[...] = a2
    b2_ref[...] = b2
    am = jnp.max(a2, axis=0, keepdims=True)
    bm = jnp.max(b2, axis=0, keepdims=True)

    @pl.when(i == 0)
    def _():
        ca_ref[...] = am
        cb_ref[...] = bm

    @pl.when(i != 0)
    def _():
        ca_ref[...] = jnp.maximum(ca_ref[...], am)
        cb_ref[...] = jnp.maximum(cb_ref[...], bm)


def _k3_body(acc_ref, bias_ref, g_ref, be_ref, o_ref):
    acc = acc_ref[...]
    s = acc[0] + acc[1]
    num = s[:, :40]
    den = s[:, 48:49]
    o = num / (den + 1e-16) + bias_ref[...]
    mu = jnp.mean(o, axis=-1, keepdims=True)
    var = jnp.mean((o - mu) ** 2, axis=-1, keepdims=True)
    o_ref[...] = (o - mu) / jnp.sqrt(var + 1e-5) * g_ref[...] + be_ref[...]


def _full(shape):
    return pl.BlockSpec(shape, lambda i: tuple(0 for _ in shape))


@jax.jit
def kernel(x, edge_index, W1, att_src1, att_dst1, bias1, gamma1, beta1, W2,
           att_src2, att_dst2, bias2, gamma2, beta2):
    src, dst = edge_index[0], edge_index[1]
    loop = jnp.arange(_N, dtype=src.dtype)
    padn = _EPAD - _ETOT
    srcp = jnp.concatenate([src, loop, jnp.zeros((padn,), src.dtype)])
    dstp = jnp.concatenate([dst, loop, jnp.full((padn,), _TRASH, dst.dtype)])

    eye8 = jnp.eye(8, dtype=_f32)
    ms = (eye8[:, None, :] * att_src1[:, :, None]).reshape(64, 8)
    ms = jnp.concatenate([ms, jnp.zeros((64, 8), _f32)], axis=1)
    md = (eye8[:, None, :] * att_dst1[:, :, None]).reshape(64, 8)
    md = jnp.concatenate([md, jnp.zeros((64, 8), _f32)], axis=1)
    kb = jnp.kron(eye8, jnp.ones((1, 8), _f32))
    w2p = jnp.concatenate([W2, jnp.zeros((64, 8), _f32)], axis=1)
    as2 = jnp.concatenate(
        [jnp.tile(att_src2.reshape(40, 1), (1, 16)), jnp.zeros((8, 16), _f32)])
    bs2 = jnp.concatenate(
        [jnp.tile(att_dst2.reshape(40, 1), (1, 16)), jnp.zeros((8, 16), _f32)])

    grid = _N // _BN
    h1, a1, b1, ca1, cb1 = pl.pallas_call(
        _k1_body,
        grid=(grid,),
        in_specs=[
            pl.BlockSpec((_BN, 128), lambda i: (i, 0)),
            _full((128, 64)),
            _full((64, 16)),
            _full((64, 16)),
        ],
        out_specs=[
            pl.BlockSpec((_BN, 64), lambda i: (i, 0)),
            pl.BlockSpec((_BN, 16), lambda i: (i, 0)),
            pl.BlockSpec((_BN, 16), lambda i: (i, 0)),
            _full((1, 16)),
            _full((1, 16)),
        ],
        out_shape=[
            jax.ShapeDtypeStruct((_N, 64), _f32),
            jax.ShapeDtypeStruct((_N, 16), _f32),
            jax.ShapeDtypeStruct((_N, 16), _f32),
            jax.ShapeDtypeStruct((1, 16), _f32),
            jax.ShapeDtypeStruct((1, 16), _f32),
        ],
    )(x, W1, ms, md)

    c1 = jax.nn.leaky_relu(ca1 + cb1, 0.2).reshape(16)
    b1p = jnp.concatenate([b1, jnp.zeros((_NPAD - _N, 16), _f32)], axis=0)
    z1 = jnp.zeros((_NPAD, 80), _f32)
    acc1 = _edge1(h1, a1, b1p, srcp, dstp, c1, z1)

    h2, a2, b2, ca2, cb2 = pl.pallas_call(
        _k2_body,
        grid=(grid,),
        in_specs=[
            pl.BlockSpec((_NC, _BN, 80), lambda i: (0, i, 0)),
            _full((8, 64)),
            _full((1, 64)),
            _full((1, 64)),
            _full((1, 64)),
            _full((64, 48)),
            _full((48, 16)),
            _full((48, 16)),
        ],
        out_specs=[
            pl.BlockSpec((_BN, 48), lambda i: (i, 0)),
            pl.BlockSpec((_BN, 16), lambda i: (i, 0)),
            pl.BlockSpec((_BN, 16), lambda i: (i, 0)),
            _full((1, 16)),
            _full((1, 16)),
        ],
        out_shape=[
            jax.ShapeDtypeStruct((_N, 48), _f32),
            jax.ShapeDtypeStruct((_N, 16), _f32),
            jax.ShapeDtypeStruct((_N, 16), _f32),
            jax.ShapeDtypeStruct((1, 16), _f32),
            jax.ShapeDtypeStruct((1, 16), _f32),
        ],
    )(acc1, kb, bias1.reshape(1, 64), gamma1.reshape(1, 64),
      beta1.reshape(1, 64), w2p, as2, bs2)

    c2 = jax.nn.leaky_relu(ca2 + cb2, 0.2).reshape(16)
    b2p = jnp.concatenate([b2, jnp.zeros((_NPAD - _N, 16), _f32)], axis=0)
    z2 = jnp.zeros((_NPAD, 64), _f32)
    acc2 = _edge2(h2, a2, b2p, srcp, dstp, c2, z2)

    out = pl.pallas_call(
        _k3_body,
        grid=(grid,),
        in_specs=[
            pl.BlockSpec((_NC, _BN, 64), lambda i: (0, i, 0)),
            _full((1, 40)),
            _full((1, 40)),
            _full((1, 40)),
        ],
        out_specs=pl.BlockSpec((_BN, 40), lambda i: (i, 0)),
        out_shape=jax.ShapeDtypeStruct((_N, 40), _f32),
    )(acc2, bias2.reshape(1, 40), gamma2.reshape(1, 40),
      beta2.reshape(1, 40))
    return out
